# baseline (speedup 1.0000x reference)
import functools

import jax
import jax.numpy as jnp
from jax import lax
from jax.experimental import pallas as pl
from jax.experimental.pallas import tpu as pltpu
from jax.experimental.pallas import tpu_sc as plsc

B = 8
N = 16384
S = 512
C = 64
W = 256
NCHUNK = N // W


def _fps_body(xyzt_ref, idx_ref, xo_ref, yo_ref, zo_ref,
              dist_ref, jgf_ref):
    x_ref = xyzt_ref.at[0]
    y_ref = xyzt_ref.at[1]
    z_ref = xyzt_ref.at[2]
    iota_s = lax.broadcasted_iota(jnp.int32, (B, S), 1)
    neg_inf = jnp.full((B, W), -jnp.inf, jnp.float32)
    zero_f = jnp.zeros((B, W), jnp.float32)
    dist_ref[:] = jnp.full((B, N), jnp.inf, jnp.float32)
    jgf_ref[:] = lax.broadcasted_iota(
        jnp.int32, (B, N), 1).astype(jnp.float32)
    def step(i, carry):
        li, qx, qy, qz = carry
        sel_s = iota_s == i
        idx_ref[:] = jnp.where(sel_s, li, idx_ref[:])
        xo_ref[:] = jnp.where(sel_s, qx, xo_ref[:])
        yo_ref[:] = jnp.where(sel_s, qy, yo_ref[:])
        zo_ref[:] = jnp.where(sel_s, qz, zo_ref[:])

        acc_m = neg_inf
        acc_j = zero_f
        acc_x = zero_f
        acc_y = zero_f
        acc_z = zero_f
        for c in range(NCHUNK):
            sl = pl.ds(c * W, W)
            xc = x_ref[:, sl]
            yc = y_ref[:, sl]
            zc = z_ref[:, sl]
            dx = xc - qx
            dy = yc - qy
            dz = zc - qz
            d = dx * dx + dy * dy + dz * dz
            dm = jnp.minimum(dist_ref[:, sl], d)
            dist_ref[:, sl] = dm
            win = dm > acc_m
            acc_m = jnp.where(win, dm, acc_m)
            acc_j = jnp.where(win, jgf_ref[:, sl], acc_j)
            acc_x = jnp.where(win, xc, acc_x)
            acc_y = jnp.where(win, yc, acc_y)
            acc_z = jnp.where(win, zc, acc_z)

        m1 = jnp.max(acc_m, axis=1, keepdims=True)
        cand = jnp.where(acc_m == m1, acc_j, jnp.float32(N))
        ni1 = jnp.min(cand, axis=1, keepdims=True)
        sel2 = cand == ni1
        nqx = jnp.sum(jnp.where(sel2, acc_x, 0.0), axis=1, keepdims=True)
        nqy = jnp.sum(jnp.where(sel2, acc_y, 0.0), axis=1, keepdims=True)
        nqz = jnp.sum(jnp.where(sel2, acc_z, 0.0), axis=1, keepdims=True)
        return ni1.astype(jnp.int32), nqx, nqy, nqz

    li0 = jnp.zeros((B, 1), jnp.int32)
    init = (li0, x_ref[:, 0:1], y_ref[:, 0:1], z_ref[:, 0:1])
    lax.fori_loop(0, S, step, init)


def _fps(xyzt):
    return pl.pallas_call(
        _fps_body,
        out_shape=[
            jax.ShapeDtypeStruct((B, S), jnp.int32),
            jax.ShapeDtypeStruct((B, S), jnp.float32),
            jax.ShapeDtypeStruct((B, S), jnp.float32),
            jax.ShapeDtypeStruct((B, S), jnp.float32),
        ],
        scratch_shapes=[pltpu.VMEM((B, N), jnp.float32),
                        pltpu.VMEM((B, N), jnp.float32)],
    )(xyzt)


_NC = 2
_NS = 16
_L = 16
_NW = _NC * _NS
_BPW = (B * S) // _NW


_PPW = (B * N) // _NW


@functools.cache
def _make_deint():
    mesh = plsc.VectorSubcoreMesh(core_axis_name="c", subcore_axis_name="s")

    @functools.partial(
        pl.kernel,
        mesh=mesh,
        out_type=jax.ShapeDtypeStruct((3, B * N), jnp.float32),
        scratch_types=[
            pltpu.VMEM((3 * _PPW,), jnp.float32),
            pltpu.VMEM((_PPW,), jnp.float32),
            pltpu.VMEM((_PPW,), jnp.float32),
            pltpu.VMEM((_PPW,), jnp.float32),
        ],
        compiler_params=pltpu.CompilerParams(use_tc_tiling_on_sc=False,
                                             needs_layout_passes=False),
    )
    def deint(src_hbm, out_hbm, buf_v, px_v, py_v, pz_v):
        wid = lax.axis_index("s") * _NC + lax.axis_index("c")
        p0 = wid * _PPW
        pltpu.sync_copy(src_hbm.at[pl.ds(3 * p0, 3 * _PPW)], buf_v)
        iota3 = lax.broadcasted_iota(jnp.int32, (_L,), 0) * 3

        def body(j, carry):
            base = j * (3 * _L)
            sl = pl.ds(j * _L, _L)
            px_v[sl] = plsc.load_gather(buf_v, [iota3 + base])
            py_v[sl] = plsc.load_gather(buf_v, [iota3 + (base + 1)])
            pz_v[sl] = plsc.load_gather(buf_v, [iota3 + (base + 2)])
            return carry

        lax.fori_loop(0, _PPW // _L, body, 0)
        pltpu.sync_copy(px_v, out_hbm.at[0, pl.ds(p0, _PPW)])
        pltpu.sync_copy(py_v, out_hbm.at[1, pl.ds(p0, _PPW)])
        pltpu.sync_copy(pz_v, out_hbm.at[2, pl.ds(p0, _PPW)])

    return deint


@functools.cache
def _make_gather():
    mesh = plsc.VectorSubcoreMesh(core_axis_name="c", subcore_axis_name="s")

    @functools.partial(
        pl.kernel,
        mesh=mesh,
        out_type=jax.ShapeDtypeStruct((B * S, C), jnp.float32),
        scratch_types=[
            pltpu.VMEM((_BPW,), jnp.int32),
            pltpu.VMEM((_BPW, C), jnp.float32),
            pltpu.SemaphoreType.DMA,
        ],
        compiler_params=pltpu.CompilerParams(use_tc_tiling_on_sc=False),
    )
    def gather_f(f_hbm, idx_hbm, out_hbm, idx_v, rows_v, sem):
        wid = lax.axis_index("s") * _NC + lax.axis_index("c")
        base = wid * _BPW
        pltpu.sync_copy(idx_hbm.at[pl.ds(base, _BPW)], idx_v)
        off = (base // S) * N
        for j in range(_BPW // _L):
            sl = pl.ds(j * _L, _L)
            idx_v[sl] = idx_v[sl] + off
        pltpu.async_copy(f_hbm.at[idx_v], rows_v, sem).wait()
        pltpu.sync_copy(rows_v, out_hbm.at[pl.ds(base, _BPW)])

    return gather_f


def kernel(xyz, f):
    xyzt = _make_deint()(xyz.reshape(B * N * 3)).reshape(3, B, N)
    idx, xo, yo, zo = _fps(xyzt)
    xyz_sampled = jnp.stack([xo, yo, zo], axis=-1)
    f_sampled = _make_gather()(f.reshape(B * N, C), idx.reshape(B * S))
    return (xyz_sampled, f_sampled.reshape(B, S, C))


# speedup vs baseline: 1.2974x; 1.2974x over previous
import functools

import jax
import jax.numpy as jnp
from jax import lax
from jax.experimental import pallas as pl
from jax.experimental.pallas import tpu as pltpu
from jax.experimental.pallas import tpu_sc as plsc

B = 8
N = 16384
S = 512
C = 64
W = 256
NCHUNK = N // W


def _fps_body(xyzt_ref, idx_ref, xo_ref, yo_ref, zo_ref,
              dist_ref, jgf_ref):
    x_ref = xyzt_ref.at[0]
    y_ref = xyzt_ref.at[1]
    z_ref = xyzt_ref.at[2]
    iota_s = lax.broadcasted_iota(jnp.int32, (B, S), 1)
    neg_inf = jnp.full((B, W), -jnp.inf, jnp.float32)
    zero_f = jnp.zeros((B, W), jnp.float32)
    dist_ref[:] = jnp.full((B, N), jnp.inf, jnp.float32)
    jgf_ref[:] = lax.broadcasted_iota(
        jnp.int32, (B, N), 1).astype(jnp.float32)
    def step(i, carry):
        li, qx, qy, qz = carry
        sel_s = iota_s == i
        idx_ref[:] = jnp.where(sel_s, li, idx_ref[:])
        xo_ref[:] = jnp.where(sel_s, qx, xo_ref[:])
        yo_ref[:] = jnp.where(sel_s, qy, yo_ref[:])
        zo_ref[:] = jnp.where(sel_s, qz, zo_ref[:])

        acc_m = neg_inf
        acc_j = zero_f
        acc_x = zero_f
        acc_y = zero_f
        acc_z = zero_f
        for c in range(NCHUNK):
            sl = pl.ds(c * W, W)
            xc = x_ref[:, sl]
            yc = y_ref[:, sl]
            zc = z_ref[:, sl]
            dx = xc - qx
            dy = yc - qy
            dz = zc - qz
            d = dx * dx + dy * dy + dz * dz
            dm = jnp.minimum(dist_ref[:, sl], d)
            dist_ref[:, sl] = dm
            win = dm > acc_m
            acc_m = jnp.where(win, dm, acc_m)
            acc_j = jnp.where(win, jgf_ref[:, sl], acc_j)
            acc_x = jnp.where(win, xc, acc_x)
            acc_y = jnp.where(win, yc, acc_y)
            acc_z = jnp.where(win, zc, acc_z)

        m1 = jnp.max(acc_m, axis=1, keepdims=True)
        cand = jnp.where(acc_m == m1, acc_j, jnp.float32(N))
        ni1 = jnp.min(cand, axis=1, keepdims=True)
        sel2 = cand == ni1
        nqx = jnp.sum(jnp.where(sel2, acc_x, 0.0), axis=1, keepdims=True)
        nqy = jnp.sum(jnp.where(sel2, acc_y, 0.0), axis=1, keepdims=True)
        nqz = jnp.sum(jnp.where(sel2, acc_z, 0.0), axis=1, keepdims=True)
        return ni1.astype(jnp.int32), nqx, nqy, nqz

    li0 = jnp.zeros((B, 1), jnp.int32)
    init = (li0, x_ref[:, 0:1], y_ref[:, 0:1], z_ref[:, 0:1])
    lax.fori_loop(0, S, step, init)


def _fps(xyzt):
    return pl.pallas_call(
        _fps_body,
        out_shape=[
            jax.ShapeDtypeStruct((B, S), jnp.int32),
            jax.ShapeDtypeStruct((B, S), jnp.float32),
            jax.ShapeDtypeStruct((B, S), jnp.float32),
            jax.ShapeDtypeStruct((B, S), jnp.float32),
        ],
        scratch_shapes=[pltpu.VMEM((B, N), jnp.float32),
                        pltpu.VMEM((B, N), jnp.float32)],
    )(xyzt)


_NC = 2
_NS = 16
_L = 16
_NW = _NC * _NS
_BPW = (B * S) // _NW


@functools.cache
def _make_gather():
    mesh = plsc.VectorSubcoreMesh(core_axis_name="c", subcore_axis_name="s")

    @functools.partial(
        pl.kernel,
        mesh=mesh,
        out_type=jax.ShapeDtypeStruct((B * S, C), jnp.float32),
        scratch_types=[
            pltpu.VMEM((_BPW,), jnp.int32),
            pltpu.VMEM((_BPW, C), jnp.float32),
            pltpu.SemaphoreType.DMA,
        ],
        compiler_params=pltpu.CompilerParams(use_tc_tiling_on_sc=False),
    )
    def gather_f(f_hbm, idx_hbm, out_hbm, idx_v, rows_v, sem):
        wid = lax.axis_index("s") * _NC + lax.axis_index("c")
        base = wid * _BPW
        pltpu.sync_copy(idx_hbm.at[pl.ds(base, _BPW)], idx_v)
        off = (base // S) * N
        for j in range(_BPW // _L):
            sl = pl.ds(j * _L, _L)
            idx_v[sl] = idx_v[sl] + off
        pltpu.async_copy(f_hbm.at[idx_v], rows_v, sem).wait()
        pltpu.sync_copy(rows_v, out_hbm.at[pl.ds(base, _BPW)])

    return gather_f


def kernel(xyz, f):
    idx, xo, yo, zo = _fps(jnp.transpose(xyz, (2, 0, 1)))
    xyz_sampled = jnp.stack([xo, yo, zo], axis=-1)
    f_sampled = _make_gather()(f.reshape(B * N, C), idx.reshape(B * S))
    return (xyz_sampled, f_sampled.reshape(B, S, C))
